# baseline (speedup 1.0000x reference)
import functools

import jax
import jax.numpy as jnp
from jax import lax
from jax.experimental import pallas as pl
from jax.experimental.pallas import tpu as pltpu
from jax.experimental.pallas import tpu_sc as plsc

NC = 2
NS = 16
NW = NC * NS

BATCH = 1024
SEQ = 200
SEQ_PAD = 256
EMBED = 64
ROWS_PER_W = BATCH // NW
SPLIT = (96, 104)
N_CHUNKS = ROWS_PER_W * 2
NBUF = 4
N_ROUNDS = N_CHUNKS // NBUF


def _body(table_hbm, idx_hbm, out_hbm, idx_v, *rest):
    bufs = rest[:NBUF]
    sems = rest[NBUF:]

    c = lax.axis_index("c")
    s = lax.axis_index("s")
    wid = s * NC + c
    row0 = wid * ROWS_PER_W

    pltpu.sync_copy(idx_hbm.at[pl.ds(row0, ROWS_PER_W)], idx_v)

    def idx_slice(j, p):
        return idx_v.at[j // 2, pl.ds(p * SPLIT[0], SPLIT[p])]

    def dst(b, p):
        return bufs[b].at[pl.ds(0, SPLIT[p])]

    def gather(j, b, p):
        pltpu.async_copy(table_hbm.at[idx_slice(j, p)], dst(b, p), sems[b])

    def wait(j, b, p):
        pltpu.make_async_copy(
            table_hbm.at[idx_slice(j, p)], dst(b, p), sems[b]
        ).wait()

    def put(j, b, p):
        base = (row0 + j // 2) * SEQ + p * SPLIT[0]
        pltpu.sync_copy(dst(b, p), out_hbm.at[pl.ds(base, SPLIT[p])])

    for b in range(NBUF):
        gather(b, b, b % 2)

    def round_body(t, carry):
        for b in range(NBUF):
            j = t * NBUF + b
            p = b % 2
            wait(j, b, p)
            put(j, b, p)
            gather(j + NBUF, b, p)
        return carry

    lax.fori_loop(0, N_ROUNDS - 1, round_body, 0)

    for b in range(NBUF):
        j = (N_ROUNDS - 1) * NBUF + b
        wait(j, b, b % 2)
        put(j, b, b % 2)


@jax.jit
def _lookup(table, idx):
    k = functools.partial(
        pl.kernel,
        out_type=jax.ShapeDtypeStruct((BATCH * SEQ, EMBED), jnp.float32),
        mesh=plsc.VectorSubcoreMesh(core_axis_name="c", subcore_axis_name="s"),
        scratch_types=[
            pltpu.VMEM((ROWS_PER_W, SEQ_PAD), jnp.int32),
            *[pltpu.VMEM((SPLIT[1], EMBED), jnp.float32) for _ in range(NBUF)],
            *[pltpu.SemaphoreType.DMA for _ in range(NBUF)],
        ],
        compiler_params=pltpu.CompilerParams(use_tc_tiling_on_sc=False),
    )(_body)
    return k(table, idx)


def kernel(token_ids, embedding_table):
    idx = jnp.pad(token_ids.astype(jnp.int32), ((0, 0), (0, SEQ_PAD - SEQ)))
    out = _lookup(embedding_table, idx)
    return out.reshape(BATCH, SEQ, EMBED)
